# baseline (speedup 1.0000x reference)
import functools

import jax
import jax.numpy as jnp
from jax import lax
from jax.experimental import pallas as pl
from jax.experimental.pallas import tpu as pltpu
from jax.experimental.pallas import tpu_sc as plsc

_CHUNK = 128


def _project_table(emb_t, wt, bias):
    d, v = emb_t.shape
    blk = 8192

    def body(e_ref, w_ref, b_ref, o_ref):
        y = (
            lax.dot_general(
                e_ref[...],
                w_ref[...],
                (((0,), (0,)), ((), ())),
                preferred_element_type=jnp.float32,
            )
            + b_ref[...]
        )
        o_ref[...] = jnp.concatenate([y, y], axis=1)

    return pl.pallas_call(
        body,
        grid=(pl.cdiv(v, blk),),
        in_specs=[
            pl.BlockSpec((d, blk), lambda i: (0, i)),
            pl.BlockSpec((d, d), lambda i: (0, 0)),
            pl.BlockSpec((1, d), lambda i: (0, 0)),
        ],
        out_specs=pl.BlockSpec((blk, 2 * d), lambda i: (i, 0)),
        out_shape=jax.ShapeDtypeStruct((v, 2 * d), jnp.float32),
    )(emb_t, wt, bias)


@functools.partial(jax.jit, static_argnums=(2,))
def _sc_gather(ids, table, chunk):
    n = ids.shape[0]
    k = table.shape[1]
    info = plsc.get_sparse_core_info()
    nc = info.num_cores
    n_workers = nc * info.num_subcores
    n_per_w = n // n_workers
    n_chunks = n_per_w // chunk

    mesh = plsc.VectorSubcoreMesh(core_axis_name="c", subcore_axis_name="s")
    nb = 4
    assert n_chunks % nb == 0 and n_chunks // nb >= 2

    @functools.partial(
        pl.kernel,
        mesh=mesh,
        out_type=jax.ShapeDtypeStruct((n, k), jnp.float32),
        scratch_types=[
            pltpu.VMEM((n_per_w,), jnp.int32),
        ]
        + [pltpu.VMEM((chunk, k), jnp.float32) for _ in range(nb)]
        + [pltpu.SemaphoreType.DMA for _ in range(2 * nb)],
    )
    def gather_kernel(idx_hbm, table_hbm, out_hbm, idx_v, *bufs_and_sems):
        rows = bufs_and_sems[:nb]
        gsem = bufs_and_sems[nb : 2 * nb]
        wsem = bufs_and_sems[2 * nb : 3 * nb]
        wid = lax.axis_index("s") * nc + lax.axis_index("c")
        base = wid * n_per_w
        pltpu.sync_copy(idx_hbm.at[pl.ds(base, n_per_w)], idx_v)

        def g_copy(i, b):
            return pltpu.make_async_copy(
                table_hbm.at[idx_v.at[pl.ds(i * chunk, chunk)]], rows[b], gsem[b]
            )

        def w_copy(i, b):
            return pltpu.make_async_copy(
                rows[b], out_hbm.at[pl.ds(base + i * chunk, chunk)], wsem[b]
            )

        for b in range(nb):
            g_copy(b, b).start()

        def body(i0, carry):
            for b in range(nb):
                i = i0 * nb + b
                g_copy(i, b).wait()
                w_copy(i, b).start()
                w_copy(i, b).wait()
                g_copy(i + nb, b).start()
            return carry

        lax.fori_loop(0, n_chunks // nb - 1, body, 0)

        for b in range(nb):
            i = n_chunks - nb + b
            g_copy(i, b).wait()
            w_copy(i, b).start()
        for b in range(nb):
            w_copy(n_chunks - nb + b, b).wait()

    return gather_kernel(ids, table)


def kernel(input_ids, emb, W, b):
    bsz, t = input_ids.shape
    v, d = emb.shape
    n = bsz * t
    ids = input_ids.reshape(n).astype(jnp.int32)
    proj = _project_table(emb.T, W.T, b.reshape(1, d))
    y2 = _sc_gather(ids, proj, _CHUNK)
    return y2[:, :d].reshape(bsz, t, d)


# speedup vs baseline: 1.0477x; 1.0477x over previous
import functools

import jax
import jax.numpy as jnp
from jax import lax
from jax.experimental import pallas as pl
from jax.experimental.pallas import tpu as pltpu
from jax.experimental.pallas import tpu_sc as plsc

_CHUNK = 128


def _project_table(emb_t, wt, bias):
    d, v = emb_t.shape
    blk = 16384

    def body(e_ref, w_ref, b_ref, o_ref):
        y = (
            lax.dot_general(
                e_ref[...],
                w_ref[...],
                (((0,), (0,)), ((), ())),
                preferred_element_type=jnp.float32,
            )
            + b_ref[...]
        )
        o_ref[...] = jnp.concatenate([y, y], axis=1)

    return pl.pallas_call(
        body,
        grid=(pl.cdiv(v, blk),),
        in_specs=[
            pl.BlockSpec((d, blk), lambda i: (0, i)),
            pl.BlockSpec((d, d), lambda i: (0, 0)),
            pl.BlockSpec((1, d), lambda i: (0, 0)),
        ],
        out_specs=pl.BlockSpec((blk, 2 * d), lambda i: (i, 0)),
        out_shape=jax.ShapeDtypeStruct((v, 2 * d), jnp.float32),
    )(emb_t, wt, bias)


@functools.partial(jax.jit, static_argnums=(2,))
def _sc_gather(ids, table, chunk):
    n = ids.shape[0]
    k = table.shape[1]
    info = plsc.get_sparse_core_info()
    nc = info.num_cores
    n_workers = nc * info.num_subcores
    n_per_w = n // n_workers
    n_chunks = n_per_w // chunk

    mesh = plsc.VectorSubcoreMesh(core_axis_name="c", subcore_axis_name="s")
    nb = 4
    assert n_chunks % nb == 0 and n_chunks // nb >= 2

    @functools.partial(
        pl.kernel,
        mesh=mesh,
        out_type=jax.ShapeDtypeStruct((n, k), jnp.float32),
        scratch_types=[
            pltpu.VMEM((n_per_w,), jnp.int32),
        ]
        + [pltpu.VMEM((chunk, k), jnp.float32) for _ in range(nb)]
        + [pltpu.SemaphoreType.DMA for _ in range(2 * nb)],
    )
    def gather_kernel(idx_hbm, table_hbm, out_hbm, idx_v, *bufs_and_sems):
        rows = bufs_and_sems[:nb]
        gsem = bufs_and_sems[nb : 2 * nb]
        wsem = bufs_and_sems[2 * nb : 3 * nb]
        wid = lax.axis_index("s") * nc + lax.axis_index("c")
        base = wid * n_per_w
        pltpu.sync_copy(idx_hbm.at[pl.ds(base, n_per_w)], idx_v)

        def g_copy(i, b):
            return pltpu.make_async_copy(
                table_hbm.at[idx_v.at[pl.ds(i * chunk, chunk)]], rows[b], gsem[b]
            )

        def w_copy(i, b):
            return pltpu.make_async_copy(
                rows[b], out_hbm.at[pl.ds(base + i * chunk, chunk)], wsem[b]
            )

        for b in range(nb):
            g_copy(b, b).start()

        def body(i0, carry):
            for b in range(nb):
                i = i0 * nb + b
                g_copy(i, b).wait()
                w_copy(i, b).start()
                w_copy(i, b).wait()
                g_copy(i + nb, b).start()
            return carry

        lax.fori_loop(0, n_chunks // nb - 1, body, 0)

        for b in range(nb):
            i = n_chunks - nb + b
            g_copy(i, b).wait()
            w_copy(i, b).start()
        for b in range(nb):
            w_copy(n_chunks - nb + b, b).wait()

    return gather_kernel(ids, table)


def kernel(input_ids, emb, W, b):
    bsz, t = input_ids.shape
    v, d = emb.shape
    n = bsz * t
    ids = input_ids.reshape(n).astype(jnp.int32)
    proj = _project_table(emb.T, W.T, b.reshape(1, d))
    y2 = _sc_gather(ids, proj, _CHUNK)
    return y2[:, :d].reshape(bsz, t, d)


# speedup vs baseline: 1.0606x; 1.0123x over previous
import functools

import jax
import jax.numpy as jnp
from jax import lax
from jax.experimental import pallas as pl
from jax.experimental.pallas import tpu as pltpu
from jax.experimental.pallas import tpu_sc as plsc

_CHUNK = 128


def _project_table(emb_t, wt, bias):
    d, v = emb_t.shape
    blk = 24576

    def body(e_ref, w_ref, b_ref, o_ref):
        y = (
            lax.dot_general(
                e_ref[...],
                w_ref[...],
                (((0,), (0,)), ((), ())),
                preferred_element_type=jnp.float32,
            )
            + b_ref[...]
        )
        o_ref[...] = jnp.concatenate([y, y], axis=1)

    return pl.pallas_call(
        body,
        grid=(pl.cdiv(v, blk),),
        in_specs=[
            pl.BlockSpec((d, blk), lambda i: (0, i)),
            pl.BlockSpec((d, d), lambda i: (0, 0)),
            pl.BlockSpec((1, d), lambda i: (0, 0)),
        ],
        out_specs=pl.BlockSpec((blk, 2 * d), lambda i: (i, 0)),
        out_shape=jax.ShapeDtypeStruct((v, 2 * d), jnp.float32),
    )(emb_t, wt, bias)


@functools.partial(jax.jit, static_argnums=(2,))
def _sc_gather(ids, table, chunk):
    n = ids.shape[0]
    k = table.shape[1]
    info = plsc.get_sparse_core_info()
    nc = info.num_cores
    n_workers = nc * info.num_subcores
    n_per_w = n // n_workers
    n_chunks = n_per_w // chunk

    mesh = plsc.VectorSubcoreMesh(core_axis_name="c", subcore_axis_name="s")
    nb = 4
    assert n_chunks % nb == 0 and n_chunks // nb >= 2

    @functools.partial(
        pl.kernel,
        mesh=mesh,
        out_type=jax.ShapeDtypeStruct((n, k), jnp.float32),
        scratch_types=[
            pltpu.VMEM((n_per_w,), jnp.int32),
        ]
        + [pltpu.VMEM((chunk, k), jnp.float32) for _ in range(nb)]
        + [pltpu.SemaphoreType.DMA for _ in range(2 * nb)],
    )
    def gather_kernel(idx_hbm, table_hbm, out_hbm, idx_v, *bufs_and_sems):
        rows = bufs_and_sems[:nb]
        gsem = bufs_and_sems[nb : 2 * nb]
        wsem = bufs_and_sems[2 * nb : 3 * nb]
        wid = lax.axis_index("s") * nc + lax.axis_index("c")
        base = wid * n_per_w
        pltpu.sync_copy(idx_hbm.at[pl.ds(base, n_per_w)], idx_v)

        def g_copy(i, b):
            return pltpu.make_async_copy(
                table_hbm.at[idx_v.at[pl.ds(i * chunk, chunk)]], rows[b], gsem[b]
            )

        def w_copy(i, b):
            return pltpu.make_async_copy(
                rows[b], out_hbm.at[pl.ds(base + i * chunk, chunk)], wsem[b]
            )

        for b in range(nb):
            g_copy(b, b).start()

        def body(i0, carry):
            for b in range(nb):
                i = i0 * nb + b
                g_copy(i, b).wait()
                w_copy(i, b).start()
                w_copy(i, b).wait()
                g_copy(i + nb, b).start()
            return carry

        lax.fori_loop(0, n_chunks // nb - 1, body, 0)

        for b in range(nb):
            i = n_chunks - nb + b
            g_copy(i, b).wait()
            w_copy(i, b).start()
        for b in range(nb):
            w_copy(n_chunks - nb + b, b).wait()

    return gather_kernel(ids, table)


def kernel(input_ids, emb, W, b):
    bsz, t = input_ids.shape
    v, d = emb.shape
    n = bsz * t
    ids = input_ids.reshape(n).astype(jnp.int32)
    proj = _project_table(emb.T, W.T, b.reshape(1, d))
    y2 = _sc_gather(ids, proj, _CHUNK)
    return y2[:, :d].reshape(bsz, t, d)


# speedup vs baseline: 1.1092x; 1.0458x over previous
import functools

import jax
import jax.numpy as jnp
from jax import lax
from jax.experimental import pallas as pl
from jax.experimental.pallas import tpu as pltpu
from jax.experimental.pallas import tpu_sc as plsc

_CHUNK = 128


def _project_table(emb_t, wt, bias):
    d, v = emb_t.shape
    blk = 24576

    def body(e_ref, w_ref, b_ref, o_ref):
        o_ref[...] = (
            lax.dot_general(
                e_ref[...],
                w_ref[...],
                (((0,), (0,)), ((), ())),
                preferred_element_type=jnp.float32,
            )
            + b_ref[...]
        )

    return pl.pallas_call(
        body,
        grid=(pl.cdiv(v, blk),),
        in_specs=[
            pl.BlockSpec((d, blk), lambda i: (0, i)),
            pl.BlockSpec((d, 2 * d), lambda i: (0, 0)),
            pl.BlockSpec((1, 2 * d), lambda i: (0, 0)),
        ],
        out_specs=pl.BlockSpec((blk, 2 * d), lambda i: (i, 0)),
        out_shape=jax.ShapeDtypeStruct((v, 2 * d), jnp.float32),
    )(emb_t, wt, bias)


@functools.partial(jax.jit, static_argnums=(2,))
def _sc_gather(ids, table, chunk):
    n = ids.shape[0]
    k = table.shape[1]
    info = plsc.get_sparse_core_info()
    nc = info.num_cores
    n_workers = nc * info.num_subcores
    n_per_w = n // n_workers
    n_chunks = n_per_w // chunk

    mesh = plsc.VectorSubcoreMesh(core_axis_name="c", subcore_axis_name="s")
    nb = 4
    assert n_chunks % nb == 0 and n_chunks // nb >= 2

    @functools.partial(
        pl.kernel,
        mesh=mesh,
        out_type=jax.ShapeDtypeStruct((n, k), jnp.float32),
        scratch_types=[
            pltpu.VMEM((n_per_w,), jnp.int32),
        ]
        + [pltpu.VMEM((chunk, k), jnp.float32) for _ in range(nb)]
        + [pltpu.SemaphoreType.DMA for _ in range(2 * nb)],
    )
    def gather_kernel(idx_hbm, table_hbm, out_hbm, idx_v, *bufs_and_sems):
        rows = bufs_and_sems[:nb]
        gsem = bufs_and_sems[nb : 2 * nb]
        wsem = bufs_and_sems[2 * nb : 3 * nb]
        wid = lax.axis_index("s") * nc + lax.axis_index("c")
        base = wid * n_per_w
        pltpu.sync_copy(idx_hbm.at[pl.ds(base, n_per_w)], idx_v)

        def g_copy(i, b):
            return pltpu.make_async_copy(
                table_hbm.at[idx_v.at[pl.ds(i * chunk, chunk)]], rows[b], gsem[b]
            )

        def w_copy(i, b):
            return pltpu.make_async_copy(
                rows[b], out_hbm.at[pl.ds(base + i * chunk, chunk)], wsem[b]
            )

        for b in range(nb):
            g_copy(b, b).start()

        def body(i0, carry):
            for b in range(nb):
                i = i0 * nb + b
                g_copy(i, b).wait()
                w_copy(i, b).start()
                w_copy(i, b).wait()
                g_copy(i + nb, b).start()
            return carry

        lax.fori_loop(0, n_chunks // nb - 1, body, 0)

        for b in range(nb):
            i = n_chunks - nb + b
            g_copy(i, b).wait()
            w_copy(i, b).start()
        for b in range(nb):
            w_copy(n_chunks - nb + b, b).wait()

    return gather_kernel(ids, table)


def kernel(input_ids, emb, W, b):
    bsz, t = input_ids.shape
    v, d = emb.shape
    n = bsz * t
    ids = input_ids.reshape(n).astype(jnp.int32)
    wt2 = jnp.concatenate([W.T, W.T], axis=1)
    b2 = jnp.concatenate([b, b]).reshape(1, 2 * d)
    proj = _project_table(emb.T, wt2, b2)
    y2 = _sc_gather(ids, proj, _CHUNK)
    return y2[:, :d].reshape(bsz, t, d)
